# baseline (device time: 213004 ns/iter reference)
import jax
import jax.numpy as jnp
from jax import lax
from jax.experimental import pallas as pl
from jax.experimental.pallas import tpu as pltpu

N_DEV = 8
K_TILE = 1024
N_RING = 3


def kernel(x, w_mat):
    m_per, k_dim = x.shape
    _, n_total = w_mat.shape
    n_per = n_total // N_DEV
    n_wide = n_total // 2
    n_k = k_dim // K_TILE
    n_steps = 2 * n_k

    my = lax.axis_index("i")
    my_half = my // 4
    my_mod = my % 4
    other_half = 1 - my_half
    rot = (my_mod + jnp.arange(4, dtype=jnp.int32)) % 4
    wcolb = jnp.stack([other_half, my_half]).astype(jnp.int32) * n_wide
    tgt = jnp.concatenate([other_half * 4 + rot, my_half * 4 + rot])
    coloff = rot * n_per

    def body(tgt_ref, coloff_ref, wcolb_ref, x_ref, w_ref, out_ref,
             acc_ref, xbuf_ref, wbuf_ref,
             xsems, wsems, send_sems, recv_sems, local_sem):
        s = pl.program_id(0)
        k = pl.program_id(1)
        me = lax.axis_index("i")
        t = s * n_k + k

        def start_x(t2):
            kd = lax.rem(t2, n_k)
            b = lax.rem(t2, N_RING)
            pltpu.make_async_copy(
                x_ref.at[:, pl.ds(kd * K_TILE, K_TILE)],
                xbuf_ref.at[b],
                xsems.at[b],
            ).start()

        def start_w(t2):
            s2 = lax.div(t2, n_k)
            kd = lax.rem(t2, n_k)
            b = lax.rem(t2, N_RING)
            pltpu.make_async_copy(
                w_ref.at[pl.ds(kd * K_TILE, K_TILE),
                         pl.ds(pl.multiple_of(wcolb_ref[s2], n_wide),
                               n_wide)],
                wbuf_ref.at[b],
                wsems.at[b],
            ).start()

        @pl.when(t == 0)
        def _entry():
            barrier = pltpu.get_barrier_semaphore()
            for p in range(N_DEV):
                pl.semaphore_signal(
                    barrier, inc=1,
                    device_id=(p,), device_id_type=pl.DeviceIdType.MESH,
                )
            pl.semaphore_wait(barrier, N_DEV)
            for t2 in range(2):
                start_x(jnp.int32(t2))
                start_w(jnp.int32(t2))

        b = lax.rem(t, N_RING)
        pltpu.make_async_copy(
            x_ref.at[:, pl.ds(0, K_TILE)], xbuf_ref.at[b], xsems.at[b],
        ).wait()
        pltpu.make_async_copy(
            w_ref.at[pl.ds(0, K_TILE), pl.ds(0, n_wide)],
            wbuf_ref.at[b], wsems.at[b],
        ).wait()

        @pl.when(t + 2 < n_steps)
        def _prefetch():
            start_x(t + 2)
            start_w(t + 2)

        prod = jnp.dot(
            xbuf_ref[b], wbuf_ref[b],
            preferred_element_type=jnp.float32,
        )

        @pl.when(k == 0)
        def _init():
            acc_ref[s] = prod

        @pl.when(k != 0)
        def _accum():
            acc_ref[s] += prod

        @pl.when((s == 1) & (k < 4))
        def _send_sb0():
            rdma = pltpu.make_async_remote_copy(
                src_ref=acc_ref.at[0, :, pl.ds(pl.multiple_of(coloff_ref[k], n_per), n_per)],
                dst_ref=out_ref.at[pl.ds(me * m_per, m_per)],
                send_sem=send_sems.at[k],
                recv_sem=recv_sems.at[me],
                device_id=(tgt_ref[k],),
                device_id_type=pl.DeviceIdType.MESH,
            )
            rdma.start()

        @pl.when(t == n_steps - 1)
        def _drain():
            pltpu.make_async_copy(
                acc_ref.at[1, :, pl.ds(pl.multiple_of(coloff_ref[0], n_per), n_per)],
                out_ref.at[pl.ds(me * m_per, m_per)],
                local_sem,
            ).start()
            for c in range(1, 4):
                rdma = pltpu.make_async_remote_copy(
                    src_ref=acc_ref.at[1, :, pl.ds(pl.multiple_of(coloff_ref[c], n_per), n_per)],
                    dst_ref=out_ref.at[pl.ds(me * m_per, m_per)],
                    send_sem=send_sems.at[4 + c],
                    recv_sem=recv_sems.at[me],
                    device_id=(tgt_ref[4 + c],),
                    device_id_type=pl.DeviceIdType.MESH,
                )
                rdma.start()
            pltpu.make_async_copy(
                acc_ref.at[1, :, pl.ds(pl.multiple_of(coloff_ref[0], n_per), n_per)],
                out_ref.at[pl.ds(me * m_per, m_per)],
                local_sem,
            ).wait()
            for p in range(N_DEV):
                @pl.when(p != me)
                def _wait_recv(p=p):
                    rdma = pltpu.make_async_remote_copy(
                        src_ref=acc_ref.at[0, :, pl.ds(0, n_per)],
                        dst_ref=out_ref.at[pl.ds(p * m_per, m_per)],
                        send_sem=send_sems.at[0],
                        recv_sem=recv_sems.at[p],
                        device_id=(p,),
                        device_id_type=pl.DeviceIdType.MESH,
                    )
                    rdma.wait_recv()
            for c in list(range(4)) + [5, 6, 7]:
                rdma = pltpu.make_async_remote_copy(
                    src_ref=acc_ref.at[c // 4, :, pl.ds(0, n_per)],
                    dst_ref=out_ref.at[pl.ds(0, m_per)],
                    send_sem=send_sems.at[c],
                    recv_sem=recv_sems.at[0],
                    device_id=(0,),
                    device_id_type=pl.DeviceIdType.MESH,
                )
                rdma.wait_send()

    return pl.pallas_call(
        body,
        grid_spec=pltpu.PrefetchScalarGridSpec(
            num_scalar_prefetch=3,
            grid=(2, n_k),
            in_specs=[
                pl.BlockSpec(memory_space=pl.ANY),
                pl.BlockSpec(memory_space=pl.ANY),
            ],
            out_specs=pl.BlockSpec(memory_space=pl.ANY),
            scratch_shapes=[
                pltpu.VMEM((2, m_per, n_wide), jnp.float32),
                pltpu.VMEM((N_RING, m_per, K_TILE), jnp.float32),
                pltpu.VMEM((N_RING, K_TILE, n_wide), jnp.float32),
                pltpu.SemaphoreType.DMA((N_RING,)),
                pltpu.SemaphoreType.DMA((N_RING,)),
                pltpu.SemaphoreType.DMA((N_DEV,)),
                pltpu.SemaphoreType.DMA((N_DEV,)),
                pltpu.SemaphoreType.DMA,
            ],
        ),
        out_shape=jax.ShapeDtypeStruct((N_DEV * m_per, n_per), jnp.float32),
        compiler_params=pltpu.CompilerParams(
            dimension_semantics=("arbitrary", "arbitrary"),
            collective_id=0,
            vmem_limit_bytes=128 * 1024 * 1024,
        ),
    )(tgt, coloff, wcolb, x, w_mat)
